# baseline (speedup 1.0000x reference)
import jax
import jax.numpy as jnp
from jax import lax
from jax.experimental import pallas as pl
from jax.experimental.pallas import tpu as pltpu, tpu_sc as plsc

EPS = 1e-6
NC = 2
NS = 16
L = 16

B = 16
A = 65536
T = 32768
BPC = B // NC
TPW = T // 2
C = 1024
NCHUNK = TPW // C
OW = 12
W = 3 * C


def _rsqrt(x):
    i = plsc.bitcast(x, jnp.int32)
    y = plsc.bitcast(jnp.int32(0x5F3759DF) - (i >> 1), jnp.float32)
    xh = x * 0.5
    y = y * (1.5 - xh * y * y)
    y = y * (1.5 - xh * y * y)
    return y


def _sqrt(x):
    return x * _rsqrt(x + 1e-35)


def _body(pts_hbm, tri_hbm, out_hbm, sh, iv, w0, w1, w2,
          pw0, pw1, pw2, ob, gsem):
    c = lax.axis_index("c")
    s = lax.axis_index("s")
    b = c * BPC + s // 2
    half = s % 2
    soff = (s // 2) * (3 * A)

    spt = BPC * A * 3 // NS
    pltpu.sync_copy(pts_hbm.at[pl.ds(c * BPC * A * 3 + s * spt, spt)],
                    sh.at[pl.ds(s * spt, spt)])
    plsc.subcore_barrier()

    iota = lax.iota(jnp.int32, L)
    oidx0 = iota * OW
    ridx0 = iota * 3

    def chunk_body(j, carry):
        start = half * TPW + j * C
        pltpu.sync_copy(tri_hbm.at[b, pl.ds(start * 3, W)], iv)

        def expand_body(i, carry2):
            o = i * L
            w = iv[pl.ds(o, L)] * 3 + soff
            w0[pl.ds(o, L)] = w
            w1[pl.ds(o, L)] = w + 1
            w2[pl.ds(o, L)] = w + 2
            return carry2

        lax.fori_loop(0, W // L, expand_body, 0)

        copies = [pltpu.async_copy(sh.at[wr], pw, gsem)
                  for wr, pw in ((w0, pw0), (w1, pw1), (w2, pw2))]
        for d in copies:
            d.wait()

        def group_body(hbase, g):
            base = hbase + g * L
            rbase = ridx0 + base * 3

            def ld(pw, k):
                return plsc.load_gather(pw, [rbase + k])

            p0x, p0y, p0z = ld(pw0, 0), ld(pw1, 0), ld(pw2, 0)
            p1x, p1y, p1z = ld(pw0, 1), ld(pw1, 1), ld(pw2, 1)
            p2x, p2y, p2z = ld(pw0, 2), ld(pw1, 2), ld(pw2, 2)

            d10x, d10y, d10z = p1x - p0x, p1y - p0y, p1z - p0z
            d20x, d20y, d20z = p2x - p0x, p2y - p0y, p2z - p0z

            s10 = d10x * d10x + d10y * d10y + d10z * d10z
            inv10 = 1.0 / (_sqrt(s10) + EPS)
            zx = d10x * inv10
            zy = d10y * inv10
            zz = (d10z + EPS) * inv10

            yrx = zy * d20z - zz * d20y
            yry = zz * d20x - zx * d20z
            yrz = zx * d20y - zy * d20x
            sy = yrx * yrx + yry * yry + yrz * yrz
            invy = 1.0 / (_sqrt(sy) + EPS)
            yx = yrx * invy
            yy = (yry + EPS) * invy
            yz = yrz * invy

            xrx = yy * zz - yz * zy
            xry = yz * zx - yx * zz
            xrz = yx * zy - yy * zx
            sx = xrx * xrx + xry * xry + xrz * xrz
            invx = 1.0 / (_sqrt(sx) + EPS)
            xx = (xrx + EPS) * invx
            xy = xry * invx
            xz = xrz * invx

            oi = oidx0 + g * (L * OW)
            comps = (p0x, p0y, p0z, xx, xy, xz, yx, yy, yz, zx, zy, zz)
            for ci, v in enumerate(comps):
                plsc.store_scatter(ob, [oi + ci], v)
            return 0

        for h in range(2):
            hbase = h * (C // 2)
            lax.fori_loop(0, C // 2 // L,
                          lambda g, cy: group_body(hbase, g), 0)
            pltpu.sync_copy(
                ob, out_hbm.at[b, pl.ds((start + hbase) * OW, C // 2 * OW)])
        return carry

    lax.fori_loop(0, NCHUNK, chunk_body, 0)


@jax.jit
def _frames(pts_flat, tri_flat):
    mesh = plsc.VectorSubcoreMesh(
        core_axis_name="c", subcore_axis_name="s",
        num_cores=NC, num_subcores=NS)
    return pl.kernel(
        _body,
        out_type=jax.ShapeDtypeStruct((B, T * OW), jnp.float32),
        mesh=mesh,
        compiler_params=pltpu.CompilerParams(needs_layout_passes=False),
        scratch_types=[
            pltpu.VMEM_SHARED((BPC * A * 3,), jnp.float32),
            pltpu.VMEM((W,), jnp.int32),
            pltpu.VMEM((W,), jnp.int32),
            pltpu.VMEM((W,), jnp.int32),
            pltpu.VMEM((W,), jnp.int32),
            pltpu.VMEM((W,), jnp.float32),
            pltpu.VMEM((W,), jnp.float32),
            pltpu.VMEM((W,), jnp.float32),
            pltpu.VMEM((C // 2 * OW,), jnp.float32),
            pltpu.SemaphoreType.DMA,
        ],
    )(pts_flat, tri_flat)


def kernel(points, triplets):
    out = _frames(points.reshape(B * A * 3), triplets.reshape(B, T * 3))
    return out.reshape(B, T, 4, 3)


# speedup vs baseline: 5.0385x; 5.0385x over previous
import jax
import jax.numpy as jnp
from jax import lax
from jax.experimental import pallas as pl
from jax.experimental.pallas import tpu as pltpu, tpu_sc as plsc

EPS = 1e-6
NC = 2
NS = 16
L = 16

B = 16
A = 65536
T = 32768
BPC = B // NC
TPW = T // 2
C = 1024
NCHUNK = TPW // C
OW = 12


def _rsqrt(x):
    i = plsc.bitcast(x, jnp.int32)
    y = plsc.bitcast(jnp.int32(0x5F3759DF) - (i >> 1), jnp.float32)
    xh = x * 0.5
    y = y * (1.5 - xh * y * y)
    y = y * (1.5 - xh * y * y)
    return y


def _sqrt(x):
    return x * _rsqrt(x + 1e-35)


def _body(xs_hbm, ys_hbm, zs_hbm, tri_hbm, out_hbm, shx, shy, shz,
          iv, w0, w1, w2, pw, ob, gsem):
    c = lax.axis_index("c")
    s = lax.axis_index("s")
    b = c * BPC + s // 2
    half = s % 2
    soff = (s // 2) * A

    rpt = BPC * A // NS
    src0 = c * BPC * A + s * rpt
    dst0 = s * rpt
    pltpu.sync_copy(xs_hbm.at[pl.ds(src0, rpt)], shx.at[pl.ds(dst0, rpt)])
    pltpu.sync_copy(ys_hbm.at[pl.ds(src0, rpt)], shy.at[pl.ds(dst0, rpt)])
    pltpu.sync_copy(zs_hbm.at[pl.ds(src0, rpt)], shz.at[pl.ds(dst0, rpt)])
    plsc.subcore_barrier()

    iota = lax.iota(jnp.int32, L)
    oidx0 = iota * OW
    ridx0 = iota * 3

    def chunk_body(j, carry):
        start = half * TPW + j * C
        pltpu.sync_copy(tri_hbm.at[b, pl.ds(start * 3, 3 * C)], iv)

        def deint_body(i, carry2):
            ri = ridx0 + i * (3 * L)
            o = i * L
            w0[pl.ds(o, L)] = plsc.load_gather(iv, [ri]) + soff
            w1[pl.ds(o, L)] = plsc.load_gather(iv, [ri + 1]) + soff
            w2[pl.ds(o, L)] = plsc.load_gather(iv, [ri + 2]) + soff
            return carry2

        lax.fori_loop(0, C // L, deint_body, 0)

        copies = []
        for k, wr in enumerate((w0, w1, w2)):
            for cc, plane in enumerate((shx, shy, shz)):
                copies.append(pltpu.async_copy(
                    plane.at[wr], pw.at[pl.ds((k * 3 + cc) * C, C)], gsem))
        for d in copies:
            d.wait()

        def group_body(g, carry3):
            base = g * L

            def ld(k, cc):
                return pw[pl.ds((k * 3 + cc) * C + base, L)]

            p0x, p0y, p0z = ld(0, 0), ld(0, 1), ld(0, 2)
            p1x, p1y, p1z = ld(1, 0), ld(1, 1), ld(1, 2)
            p2x, p2y, p2z = ld(2, 0), ld(2, 1), ld(2, 2)

            d10x, d10y, d10z = p1x - p0x, p1y - p0y, p1z - p0z
            d20x, d20y, d20z = p2x - p0x, p2y - p0y, p2z - p0z

            s10 = d10x * d10x + d10y * d10y + d10z * d10z
            inv10 = 1.0 / (_sqrt(s10) + EPS)
            zx = d10x * inv10
            zy = d10y * inv10
            zz = (d10z + EPS) * inv10

            yrx = zy * d20z - zz * d20y
            yry = zz * d20x - zx * d20z
            yrz = zx * d20y - zy * d20x
            sy = yrx * yrx + yry * yry + yrz * yrz
            invy = 1.0 / (_sqrt(sy) + EPS)
            yx = yrx * invy
            yy = (yry + EPS) * invy
            yz = yrz * invy

            xrx = yy * zz - yz * zy
            xry = yz * zx - yx * zz
            xrz = yx * zy - yy * zx
            sx = xrx * xrx + xry * xry + xrz * xrz
            invx = 1.0 / (_sqrt(sx) + EPS)
            xx = (xrx + EPS) * invx
            xy = xry * invx
            xz = xrz * invx

            oi = oidx0 + base * OW
            comps = (p0x, p0y, p0z, xx, xy, xz, yx, yy, yz, zx, zy, zz)
            for ci, v in enumerate(comps):
                plsc.store_scatter(ob, [oi + ci], v)
            return carry3

        lax.fori_loop(0, C // L, group_body, 0)
        pltpu.sync_copy(ob, out_hbm.at[b, pl.ds(start * OW, C * OW)])
        return carry

    lax.fori_loop(0, NCHUNK, chunk_body, 0)


@jax.jit
def _frames(xs, ys, zs, tri_flat):
    mesh = plsc.VectorSubcoreMesh(
        core_axis_name="c", subcore_axis_name="s",
        num_cores=NC, num_subcores=NS)
    return pl.kernel(
        _body,
        out_type=jax.ShapeDtypeStruct((B, T * OW), jnp.float32),
        mesh=mesh,
        compiler_params=pltpu.CompilerParams(needs_layout_passes=False),
        scratch_types=[
            pltpu.VMEM_SHARED((BPC * A,), jnp.float32),
            pltpu.VMEM_SHARED((BPC * A,), jnp.float32),
            pltpu.VMEM_SHARED((BPC * A,), jnp.float32),
            pltpu.VMEM((3 * C,), jnp.int32),
            pltpu.VMEM((C,), jnp.int32),
            pltpu.VMEM((C,), jnp.int32),
            pltpu.VMEM((C,), jnp.int32),
            pltpu.VMEM((9 * C,), jnp.float32),
            pltpu.VMEM((C * OW,), jnp.float32),
            pltpu.SemaphoreType.DMA,
        ],
    )(xs, ys, zs, tri_flat)


def kernel(points, triplets):
    planes = points.transpose(2, 0, 1).reshape(3, B * A)
    out = _frames(planes[0], planes[1], planes[2], triplets.reshape(B, T * 3))
    return out.reshape(B, T, 4, 3)


# speedup vs baseline: 5.5222x; 1.0960x over previous
import jax
import jax.numpy as jnp
from jax import lax
from jax.experimental import pallas as pl
from jax.experimental.pallas import tpu as pltpu, tpu_sc as plsc

EPS = 1e-6
NC = 2
NS = 16
L = 16

B = 16
A = 65536
T = 32768
BPC = B // NC
TPW = T // 2
C = 1024
NCHUNK = TPW // C
OW = 12


def _rsqrt(x):
    i = plsc.bitcast(x, jnp.int32)
    y = plsc.bitcast(jnp.int32(0x5F3759DF) - (i >> 1), jnp.float32)
    xh = x * 0.5
    y = y * (1.5 - xh * y * y)
    y = y * (1.5 - xh * y * y)
    y = y * (1.5 - xh * y * y)
    return y


def _sqrt(x):
    return x * _rsqrt(x + 1e-35)


def _body(xs_hbm, ys_hbm, zs_hbm, tri_hbm, out_hbm,
          shx, shy, shz, iv, pv, ob, sem):
    c = lax.axis_index("c")
    s = lax.axis_index("s")
    b = c * BPC + s // 2
    half = s % 2

    rpt = BPC * A // NS
    src0 = c * BPC * A + s * rpt
    dst0 = s * rpt
    pltpu.sync_copy(xs_hbm.at[pl.ds(src0, rpt)], shx.at[pl.ds(dst0, rpt)])
    pltpu.sync_copy(ys_hbm.at[pl.ds(src0, rpt)], shy.at[pl.ds(dst0, rpt)])
    pltpu.sync_copy(zs_hbm.at[pl.ds(src0, rpt)], shz.at[pl.ds(dst0, rpt)])
    plsc.subcore_barrier()

    iota = lax.iota(jnp.int32, L)
    oidx0 = iota * OW

    def chunk_body(i, carry):
        start = half * TPW + i * C
        pltpu.sync_copy(
            tri_hbm.at[b, pl.ds((half * NCHUNK + i) * 3 * C, 3 * C)], iv)
        copies = []
        for k in range(3):
            for comp, plane in enumerate((shx, shy, shz)):
                r = 3 * k + comp
                copies.append(pltpu.async_copy(
                    plane.at[iv.at[pl.ds(k * C, C)]],
                    pv.at[pl.ds(r * C, C)], sem))
        for d in copies:
            d.wait()

        def group_body(g, carry2):
            base = g * L

            def ld(r):
                return pv[pl.ds(r * C + base, L)]

            p0x, p0y, p0z = ld(0), ld(1), ld(2)
            p1x, p1y, p1z = ld(3), ld(4), ld(5)
            p2x, p2y, p2z = ld(6), ld(7), ld(8)

            d10x, d10y, d10z = p1x - p0x, p1y - p0y, p1z - p0z
            d20x, d20y, d20z = p2x - p0x, p2y - p0y, p2z - p0z

            s10 = d10x * d10x + d10y * d10y + d10z * d10z
            inv10 = 1.0 / (_sqrt(s10) + EPS)
            zx = d10x * inv10
            zy = d10y * inv10
            zz = (d10z + EPS) * inv10

            yrx = zy * d20z - zz * d20y
            yry = zz * d20x - zx * d20z
            yrz = zx * d20y - zy * d20x
            sy = yrx * yrx + yry * yry + yrz * yrz
            invy = 1.0 / (_sqrt(sy) + EPS)
            yx = yrx * invy
            yy = (yry + EPS) * invy
            yz = yrz * invy

            xrx = yy * zz - yz * zy
            xry = yz * zx - yx * zz
            xrz = yx * zy - yy * zx
            sx = xrx * xrx + xry * xry + xrz * xrz
            invx = 1.0 / (_sqrt(sx) + EPS)
            xx = (xrx + EPS) * invx
            xy = xry * invx
            xz = xrz * invx

            oi = oidx0 + base * OW
            comps = (p0x, p0y, p0z, xx, xy, xz, yx, yy, yz, zx, zy, zz)
            for ci, v in enumerate(comps):
                plsc.store_scatter(ob, [oi + ci], v)
            return carry2

        lax.fori_loop(0, C // L, group_body, 0)
        pltpu.sync_copy(ob, out_hbm.at[b, pl.ds(start * OW, C * OW)])
        return carry

    lax.fori_loop(0, NCHUNK, chunk_body, 0)


@jax.jit
def _frames(xs, ys, zs, tri_adj):
    mesh = plsc.VectorSubcoreMesh(
        core_axis_name="c", subcore_axis_name="s",
        num_cores=NC, num_subcores=NS)
    return pl.kernel(
        _body,
        out_type=jax.ShapeDtypeStruct((B, T * OW), jnp.float32),
        mesh=mesh,
        compiler_params=pltpu.CompilerParams(needs_layout_passes=False),
        scratch_types=[
            pltpu.VMEM_SHARED((BPC * A,), jnp.float32),
            pltpu.VMEM_SHARED((BPC * A,), jnp.float32),
            pltpu.VMEM_SHARED((BPC * A,), jnp.float32),
            pltpu.VMEM((3 * C,), jnp.int32),
            pltpu.VMEM((9 * C,), jnp.float32),
            pltpu.VMEM((C * OW,), jnp.float32),
            pltpu.SemaphoreType.DMA,
        ],
    )(xs, ys, zs, tri_adj)


def kernel(points, triplets):
    n_atoms = points.shape[-2]
    tri = jnp.clip(triplets, 0, n_atoms - 1).astype(jnp.int32)
    tri = tri.transpose(0, 2, 1)
    slot = (jnp.arange(B, dtype=jnp.int32) % BPC) * A
    tri = tri + slot[:, None, None]
    tri = tri.reshape(B, 3, 2 * NCHUNK, C).transpose(0, 2, 1, 3).reshape(B, 3 * T)
    planes = points.transpose(2, 0, 1).reshape(3, B * A)
    out = _frames(planes[0], planes[1], planes[2], tri)
    return out.reshape(B, T, 4, 3)
